# baseline (speedup 1.0000x reference)
import jax
import jax.numpy as jnp
from jax.experimental import pallas as pl
from jax.experimental.pallas import tpu as pltpu

_B, _N, _K = 64, 196, 4096
_ROWS = _B * _N
_BLK = 128
_GRID = _ROWS // _BLK


def _dense_kernel(mask_ref, ps_ref, pt_ref, num_ref, den_ref):
    i = pl.program_id(0)
    ps = ps_ref[0]
    pt = pt_ref[0]
    m = mask_ref[0]
    safe = jnp.where(m > 0.0, ps, jnp.ones_like(ps))
    part = jnp.sum(pt * jnp.log(safe) * m)

    @pl.when(i == 0)
    def _():
        num_ref[...] = jnp.zeros_like(num_ref)
        den_ref[...] = jnp.zeros_like(den_ref)

    num_ref[...] += (-part).reshape(1, 1)
    den_ref[...] += jnp.sum(m).reshape(1, 1)


def kernel(ps, pt, bool_masked_pos):
    psf = ps.reshape(_GRID, _BLK, _K)
    ptf = pt.reshape(_GRID, _BLK, _K)
    maskf = bool_masked_pos.astype(jnp.float32).reshape(_GRID, _BLK, 1)
    num, den = pl.pallas_call(
        _dense_kernel,
        grid=(_GRID,),
        in_specs=[
            pl.BlockSpec((1, _BLK, 1), lambda i: (i, 0, 0)),
            pl.BlockSpec((1, _BLK, _K), lambda i: (i, 0, 0)),
            pl.BlockSpec((1, _BLK, _K), lambda i: (i, 0, 0)),
        ],
        out_specs=[
            pl.BlockSpec((1, 1), lambda i: (0, 0)),
            pl.BlockSpec((1, 1), lambda i: (0, 0)),
        ],
        out_shape=[
            jax.ShapeDtypeStruct((1, 1), jnp.float32),
            jax.ShapeDtypeStruct((1, 1), jnp.float32),
        ],
        compiler_params=pltpu.CompilerParams(
            dimension_semantics=("arbitrary",),
        ),
    )(maskf, psf, ptf)
    return num[0, 0] / den[0, 0]


# speedup vs baseline: 1.7430x; 1.7430x over previous
import jax
import jax.numpy as jnp
from jax.experimental import pallas as pl
from jax.experimental.pallas import tpu as pltpu

_B, _N, _K = 64, 196, 4096


def _dense_kernel(mask_ref, ps_ref, pt_ref, num_ref, den_ref):
    i = pl.program_id(0)
    ps = ps_ref[0]
    pt = pt_ref[0]
    m = mask_ref[0]
    safe = jnp.where(m > 0.0, ps, jnp.ones_like(ps))
    part = jnp.sum(pt * jnp.log(safe) * m)

    @pl.when(i == 0)
    def _():
        num_ref[...] = jnp.zeros_like(num_ref)
        den_ref[...] = jnp.zeros_like(den_ref)

    num_ref[...] += (-part).reshape(1, 1)
    den_ref[...] += jnp.sum(m).reshape(1, 1)


def kernel(ps, pt, bool_masked_pos):
    maskf = bool_masked_pos.astype(jnp.float32)[..., None]
    num, den = pl.pallas_call(
        _dense_kernel,
        grid=(_B,),
        in_specs=[
            pl.BlockSpec((1, _N, 1), lambda i: (i, 0, 0)),
            pl.BlockSpec((1, _N, _K), lambda i: (i, 0, 0)),
            pl.BlockSpec((1, _N, _K), lambda i: (i, 0, 0)),
        ],
        out_specs=[
            pl.BlockSpec((1, 1), lambda i: (0, 0)),
            pl.BlockSpec((1, 1), lambda i: (0, 0)),
        ],
        out_shape=[
            jax.ShapeDtypeStruct((1, 1), jnp.float32),
            jax.ShapeDtypeStruct((1, 1), jnp.float32),
        ],
        compiler_params=pltpu.CompilerParams(
            dimension_semantics=("arbitrary",),
        ),
    )(maskf, ps, pt)
    return num[0, 0] / den[0, 0]


# speedup vs baseline: 1.7803x; 1.0214x over previous
import jax
import jax.numpy as jnp
from jax.experimental import pallas as pl
from jax.experimental.pallas import tpu as pltpu

_B, _N, _K = 64, 196, 4096
_A = 4
_GRID = _B // _A


def _dense_kernel(*refs):
    mask_refs = refs[:_A]
    ps_refs = refs[_A:2 * _A]
    pt_refs = refs[2 * _A:3 * _A]
    num_ref, den_ref = refs[3 * _A:]
    i = pl.program_id(0)

    part = jnp.float32(0.0)
    cnt = jnp.float32(0.0)
    for j in range(_A):
        ps = ps_refs[j][0]
        pt = pt_refs[j][0]
        m = mask_refs[j][0]
        safe = jnp.where(m > 0.0, ps, jnp.ones_like(ps))
        part += jnp.sum(pt * jnp.log(safe) * m)
        cnt += jnp.sum(m)

    @pl.when(i == 0)
    def _():
        num_ref[...] = jnp.zeros_like(num_ref)
        den_ref[...] = jnp.zeros_like(den_ref)

    num_ref[...] += (-part).reshape(1, 1)
    den_ref[...] += cnt.reshape(1, 1)


def _mk_index_map(j):
    return lambda i: (_A * i + j, 0, 0)


def kernel(ps, pt, bool_masked_pos):
    maskf = bool_masked_pos.astype(jnp.float32)[..., None]
    mask_specs = [pl.BlockSpec((1, _N, 1), _mk_index_map(j)) for j in range(_A)]
    big_specs = [pl.BlockSpec((1, _N, _K), _mk_index_map(j)) for j in range(_A)]
    num, den = pl.pallas_call(
        _dense_kernel,
        grid=(_GRID,),
        in_specs=mask_specs + big_specs + big_specs,
        out_specs=[
            pl.BlockSpec((1, 1), lambda i: (0, 0)),
            pl.BlockSpec((1, 1), lambda i: (0, 0)),
        ],
        out_shape=[
            jax.ShapeDtypeStruct((1, 1), jnp.float32),
            jax.ShapeDtypeStruct((1, 1), jnp.float32),
        ],
        compiler_params=pltpu.CompilerParams(
            dimension_semantics=("arbitrary",),
        ),
    )(*([maskf] * _A), *([ps] * _A), *([pt] * _A))
    return num[0, 0] / den[0, 0]
